# baseline (device time: 429450 ns/iter reference)
import jax
import jax.numpy as jnp
from jax import lax
from jax.experimental import pallas as pl
from jax.experimental.pallas import tpu as pltpu

N_DEV = 16
M_PER = 256
N_COLS = 2048


def kernel(x, w_mat):
    m_glob, k_per = x.shape
    _, n = w_mat.shape

    def body(x_ref, w_ref, out_ref, xb_ref, wb_ref, comm_ref,
             send_sems, recv_sems, credit_sem,
             amax_ref, stage_ref, b_send_sems, b_recv_sems):
        my = lax.axis_index("i")
        left = lax.rem(my - 1 + N_DEV, N_DEV)
        right = lax.rem(my + 1, N_DEV)

        barrier = pltpu.get_barrier_semaphore()
        for nbr in (left, right):
            pl.semaphore_signal(barrier, inc=1, device_id=(nbr,),
                                device_id_type=pl.DeviceIdType.MESH)
        pl.semaphore_wait(barrier, 2)

        xb_ref[...] = x_ref[...].astype(jnp.bfloat16)
        wb_ref[...] = w_ref[...].astype(jnp.bfloat16)

        def partial_chunk(c):
            return jnp.dot(xb_ref[pl.ds(c * M_PER, M_PER), :], wb_ref[...],
                           preferred_element_type=jnp.float32)

        for s in range(N_DEV - 1):
            c = lax.rem(my - 1 - s + 2 * N_DEV, N_DEV)
            slot = s % 2
            p = partial_chunk(c)
            if s == 0:
                comm_ref[0] = p
            else:
                comm_ref[slot] = comm_ref[slot] + p
            if s >= 1:
                pl.semaphore_wait(credit_sem, 1)
            rdma = pltpu.make_async_remote_copy(
                src_ref=comm_ref.at[slot],
                dst_ref=comm_ref.at[(s + 1) % 2],
                send_sem=send_sems.at[slot],
                recv_sem=recv_sems.at[(s + 1) % 2],
                device_id=(right,),
                device_id_type=pl.DeviceIdType.MESH,
            )
            rdma.start()
            rdma.wait()
            if s <= N_DEV - 3:
                pl.semaphore_signal(credit_sem, inc=1, device_id=(left,),
                                    device_id_type=pl.DeviceIdType.MESH)

        acc = comm_ref[(N_DEV - 1) % 2] + partial_chunk(my)
        y = jnp.maximum(acc, 0.0)

        amax_ref[...] = jnp.full((8, 128), jnp.max(y), dtype=jnp.float32)
        for k in range(4):
            partner = lax.bitwise_xor(my, 1 << k)
            ex = pltpu.make_async_remote_copy(
                src_ref=amax_ref,
                dst_ref=stage_ref.at[k],
                send_sem=b_send_sems.at[k],
                recv_sem=b_recv_sems.at[k],
                device_id=(partner,),
                device_id_type=pl.DeviceIdType.MESH,
            )
            ex.start()
            ex.wait()
            amax_ref[...] = jnp.maximum(amax_ref[...], stage_ref[k])

        scale = amax_ref[0, 0] / 127.0
        q = jnp.clip(jnp.round(y / scale), -127.0, 127.0)
        out_ref[...] = q * scale

    return pl.pallas_call(
        body,
        out_shape=jax.ShapeDtypeStruct((M_PER, n), jnp.float32),
        in_specs=[pl.BlockSpec(memory_space=pltpu.VMEM),
                  pl.BlockSpec(memory_space=pltpu.VMEM)],
        out_specs=pl.BlockSpec(memory_space=pltpu.VMEM),
        scratch_shapes=[
            pltpu.VMEM((m_glob, k_per), jnp.bfloat16),
            pltpu.VMEM((k_per, N_COLS), jnp.bfloat16),
            pltpu.VMEM((2, M_PER, N_COLS), jnp.float32),
            pltpu.SemaphoreType.DMA((2,)),
            pltpu.SemaphoreType.DMA((2,)),
            pltpu.SemaphoreType.REGULAR,
            pltpu.VMEM((8, 128), jnp.float32),
            pltpu.VMEM((4, 8, 128), jnp.float32),
            pltpu.SemaphoreType.DMA((4,)),
            pltpu.SemaphoreType.DMA((4,)),
        ],
        compiler_params=pltpu.CompilerParams(collective_id=0),
    )(x, w_mat)


# device time: 180108 ns/iter; 2.3844x vs baseline; 2.3844x over previous
import jax
import jax.numpy as jnp
from jax import lax
from jax.experimental import pallas as pl
from jax.experimental.pallas import tpu as pltpu

N_DEV = 16
M_PER = 256
N_COLS = 2048


def kernel(x, w_mat):
    m_glob, k_per = x.shape
    _, n = w_mat.shape

    half = N_COLS // 2

    def body(x_ref, w_ref, out_ref, xb_ref, wb_ref,
             cw_ref, ccw_ref,
             cw_send, cw_recv, ccw_send, ccw_recv,
             credit_cw, credit_ccw,
             amax_ref, stage_ref, b_send_sems, b_recv_sems):
        my = lax.axis_index("i")
        left = lax.rem(my - 1 + N_DEV, N_DEV)
        right = lax.rem(my + 1, N_DEV)

        barrier = pltpu.get_barrier_semaphore()
        for nbr in (left, right):
            pl.semaphore_signal(barrier, inc=1, device_id=(nbr,),
                                device_id_type=pl.DeviceIdType.MESH)
        pl.semaphore_wait(barrier, 2)

        xb_ref[...] = x_ref[...].astype(jnp.bfloat16)
        wb_ref[...] = w_ref[...].astype(jnp.bfloat16)

        def partial(c, lo):
            return jnp.dot(xb_ref[pl.ds(c * M_PER, M_PER), :],
                           wb_ref[:, lo:lo + half],
                           preferred_element_type=jnp.float32)

        for s in range(N_DEV - 1):
            c_cw = lax.rem(my - 1 - s + 2 * N_DEV, N_DEV)
            c_ccw = lax.rem(my + 1 + s, N_DEV)
            slot = s % 2
            nxt = (s + 1) % 2
            p_cw = partial(c_cw, 0)
            p_ccw = partial(c_ccw, half)
            if s == 0:
                cw_ref[0] = p_cw.astype(jnp.bfloat16)
                ccw_ref[0] = p_ccw.astype(jnp.bfloat16)
            else:
                cw_ref[slot] = (cw_ref[slot].astype(jnp.float32)
                                + p_cw).astype(jnp.bfloat16)
                ccw_ref[slot] = (ccw_ref[slot].astype(jnp.float32)
                                 + p_ccw).astype(jnp.bfloat16)
            if s >= 1:
                pl.semaphore_wait(credit_cw, 1)
                pl.semaphore_wait(credit_ccw, 1)
            rdma_cw = pltpu.make_async_remote_copy(
                src_ref=cw_ref.at[slot], dst_ref=cw_ref.at[nxt],
                send_sem=cw_send.at[slot], recv_sem=cw_recv.at[nxt],
                device_id=(right,), device_id_type=pl.DeviceIdType.MESH,
            )
            rdma_ccw = pltpu.make_async_remote_copy(
                src_ref=ccw_ref.at[slot], dst_ref=ccw_ref.at[nxt],
                send_sem=ccw_send.at[slot], recv_sem=ccw_recv.at[nxt],
                device_id=(left,), device_id_type=pl.DeviceIdType.MESH,
            )
            rdma_cw.start()
            rdma_ccw.start()
            rdma_cw.wait()
            rdma_ccw.wait()
            if s <= N_DEV - 3:
                pl.semaphore_signal(credit_cw, inc=1, device_id=(left,),
                                    device_id_type=pl.DeviceIdType.MESH)
                pl.semaphore_signal(credit_ccw, inc=1, device_id=(right,),
                                    device_id_type=pl.DeviceIdType.MESH)

        last = (N_DEV - 1) % 2
        y_l = cw_ref[last].astype(jnp.float32) + partial(my, 0)
        y_r = ccw_ref[last].astype(jnp.float32) + partial(my, half)
        y = jnp.maximum(jnp.concatenate([y_l, y_r], axis=1), 0.0)

        amax_ref[...] = jnp.full((8, 128), jnp.max(y), dtype=jnp.float32)
        for k in range(4):
            partner = lax.bitwise_xor(my, 1 << k)
            ex = pltpu.make_async_remote_copy(
                src_ref=amax_ref,
                dst_ref=stage_ref.at[k],
                send_sem=b_send_sems.at[k],
                recv_sem=b_recv_sems.at[k],
                device_id=(partner,),
                device_id_type=pl.DeviceIdType.MESH,
            )
            ex.start()
            ex.wait()
            amax_ref[...] = jnp.maximum(amax_ref[...], stage_ref[k])

        scale = amax_ref[0, 0] / 127.0
        q = jnp.clip(jnp.round(y / scale), -127.0, 127.0)
        out_ref[...] = q * scale

    return pl.pallas_call(
        body,
        out_shape=jax.ShapeDtypeStruct((M_PER, n), jnp.float32),
        in_specs=[pl.BlockSpec(memory_space=pltpu.VMEM),
                  pl.BlockSpec(memory_space=pltpu.VMEM)],
        out_specs=pl.BlockSpec(memory_space=pltpu.VMEM),
        scratch_shapes=[
            pltpu.VMEM((m_glob, k_per), jnp.bfloat16),
            pltpu.VMEM((k_per, N_COLS), jnp.bfloat16),
            pltpu.VMEM((2, M_PER, N_COLS // 2), jnp.bfloat16),
            pltpu.VMEM((2, M_PER, N_COLS // 2), jnp.bfloat16),
            pltpu.SemaphoreType.DMA((2,)),
            pltpu.SemaphoreType.DMA((2,)),
            pltpu.SemaphoreType.DMA((2,)),
            pltpu.SemaphoreType.DMA((2,)),
            pltpu.SemaphoreType.REGULAR,
            pltpu.SemaphoreType.REGULAR,
            pltpu.VMEM((8, 128), jnp.float32),
            pltpu.VMEM((4, 8, 128), jnp.float32),
            pltpu.SemaphoreType.DMA((4,)),
            pltpu.SemaphoreType.DMA((4,)),
        ],
        compiler_params=pltpu.CompilerParams(collective_id=0),
    )(x, w_mat)
